# baseline (speedup 1.0000x reference)
import jax
import jax.numpy as jnp
from jax import lax
from jax.experimental import pallas as pl
from jax.experimental.pallas import tpu as pltpu
from jax.experimental.pallas import tpu_sc as plsc

N_USERS = 25000
N_ITEMS = 25000
N = N_USERS + N_ITEMS
HALF = 25000
HALF_PAD = 25600
NT = 2 * HALF_PAD
D = 64
NC, NS = 2, 16
SUB = 96
GP = 3
NSL = 3
IN_W = 128
IN_CPT = 400
E_PAD = IN_CPT * IN_W * NS
FLUSH = 864
FIFO = FLUSH + SUB + 48
TRASH = FIFO - 16
CAP = 52704
OSZ = NC * NS * CAP
ROWS_PER_TILE = HALF_PAD // NS
ZCH = 64
NZC = ROWS_PER_TILE // ZCH


def _scale_chunk(rows, vbv, vbase, p):
    def mg(g8, _):
        vv = vbv[pl.ds(vbase + g8 * 16, 16)]
        for l in range(16):
            e = p * SUB + g8 * 16 + l
            v = vv[l]
            for o in (0, 16, 32, 48):
                rows[e, pl.ds(o, 16)] = rows[e, pl.ds(o, 16)] * v
        return 0
    lax.fori_loop(0, SUB // 16, mg, 0)


def _zero_acc(acc, rows, s):
    def zr(i, _):
        for o in (0, 16, 32, 48):
            rows[i, pl.ds(o, 16)] = jnp.zeros((16,), jnp.float32)
        return 0
    lax.fori_loop(0, ZCH, zr, 0)

    def zc(z, _):
        pltpu.sync_copy(rows.at[pl.ds(0, ZCH)],
                        acc.at[pl.ds(s * ROWS_PER_TILE + z * ZCH, ZCH)])
        return 0
    lax.fori_loop(0, NZC, zc, 0)


def _edges_phase(tab, osrc, odst, oval, cntv, acc,
                 sbv, dbv, vbv, dx, rows, gsem, ssem, esem, base_out):
    total = cntv[pl.ds(0, 16)][0]
    ngroups = total // (GP * SUB)
    base_out = pl.multiple_of(base_out, 8)

    def gather(sl, idx_off):
        pltpu.async_copy(tab.at[sbv.at[pl.ds(idx_off, SUB)]],
                         rows.at[pl.ds(sl * SUB, SUB)], gsem[sl])

    def gather_wait(sl, idx_off):
        pltpu.make_async_copy(tab.at[sbv.at[pl.ds(idx_off, SUB)]],
                              rows.at[pl.ds(sl * SUB, SUB)], gsem[sl]).wait()

    def scatter(sl):
        pltpu.async_copy(rows.at[pl.ds(sl * SUB, SUB)],
                         acc.at[dx[sl]], ssem[sl], add=True)

    def scatter_wait(sl):
        pltpu.make_async_copy(rows.at[pl.ds(sl * SUB, SUB)],
                              acc.at[dx[sl]], ssem[sl]).wait()

    def copy_dst(sl, doff):
        def mk(u, _):
            dx[sl][pl.ds(u * 16, 16)] = dbv[pl.ds(doff + u * 16, 16)]
            return 0
        lax.fori_loop(0, SUB // 16, mk, 0)

    @pl.when(ngroups > 0)
    def _run():
        pltpu.sync_copy(osrc.at[pl.ds(base_out, GP * SUB)],
                        sbv.at[pl.ds(0, GP * SUB)])
        pltpu.sync_copy(odst.at[pl.ds(base_out, GP * SUB)],
                        dbv.at[pl.ds(0, GP * SUB)])
        pltpu.sync_copy(oval.at[pl.ds(base_out, GP * SUB)],
                        vbv.at[pl.ds(0, GP * SUB)])
        for p in (0, 1):
            gather(p, p * SUB)

        def grp(g, _):
            par = g % 2
            npar = (g + 1) % 2
            hb = par * GP * SUB
            nhb = npar * GP * SUB
            nrow = pl.multiple_of(base_out + (g + 1) * GP * SUB, 8)

            @pl.when(g + 1 < ngroups)
            def _stage():
                pltpu.async_copy(osrc.at[pl.ds(nrow, GP * SUB)],
                                 sbv.at[pl.ds(nhb, GP * SUB)], esem)
                pltpu.async_copy(odst.at[pl.ds(nrow, GP * SUB)],
                                 dbv.at[pl.ds(nhb, GP * SUB)], esem)
                pltpu.async_copy(oval.at[pl.ds(nrow, GP * SUB)],
                                 vbv.at[pl.ds(nhb, GP * SUB)], esem)

            for p in range(GP):
                sl2 = (p + 2) % NSL
                if p == 0:
                    @pl.when(g >= 1)
                    def _drain0():
                        scatter_wait(sl2)
                    gather(sl2, hb + 2 * SUB)
                else:
                    @pl.when(g + 1 < ngroups)
                    def _pref():
                        scatter_wait(sl2)
                        if p == 1:
                            pltpu.make_async_copy(
                                osrc.at[pl.ds(nrow, GP * SUB)],
                                sbv.at[pl.ds(nhb, GP * SUB)], esem).wait()
                            pltpu.make_async_copy(
                                odst.at[pl.ds(nrow, GP * SUB)],
                                dbv.at[pl.ds(nhb, GP * SUB)], esem).wait()
                            pltpu.make_async_copy(
                                oval.at[pl.ds(nrow, GP * SUB)],
                                vbv.at[pl.ds(nhb, GP * SUB)], esem).wait()
                        gather(sl2, nhb + (p - 1) * SUB)

                gather_wait(p, hb + p * SUB)
                _scale_chunk(rows, vbv, hb + p * SUB, p)
                copy_dst(p, hb + p * SUB)
                scatter(p)
            return 0
        lax.fori_loop(0, ngroups, grp, 0)
        for sl in range(NSL):
            scatter_wait(sl)


def _writeback(acc, dst_tab, c, s):
    pltpu.sync_copy(
        acc.at[pl.ds(s * ROWS_PER_TILE, ROWS_PER_TILE)],
        dst_tab.at[pl.ds(c * HALF_PAD + s * ROWS_PER_TILE, ROWS_PER_TILE)])


def _body1(ego_tab, srcr, dstr, valr,
           t1, osrc, odst, oval, counts,
           acc, sb, db, vb, fs, fd, fv,
           sbv, dbv, vbv, dx0, dx1, dx2, rows, cntv,
           gs0, gs1, gs2, ss0, ss1, ss2, esem):
    c = lax.axis_index("c")
    s = lax.axis_index("s")
    tid = c * NS + s
    base_out = pl.multiple_of(tid * CAP, 8)
    dx = [dx0, dx1, dx2]
    gsem = [gs0, gs1, gs2]
    ssem = [ss0, ss1, ss2]
    dlo = c * HALF
    dhi = dlo + HALF

    def block(b, carry):
        pos, total = carry
        row0 = s * IN_CPT + b * 8
        pltpu.sync_copy(srcr.at[pl.ds(row0, 8)], sb)
        pltpu.sync_copy(dstr.at[pl.ds(row0, 8)], db)
        pltpu.sync_copy(valr.at[pl.ds(row0, 8)], vb)

        iota16 = lax.iota(jnp.int32, 16)

        def chunk(r, carry2):
            pos2, total2 = carry2

            def g16(u, pos3):
                sv = sb[r, pl.ds(u * 16, 16)]
                dv = db[r, pl.ds(u * 16, 16)]
                vv = vb[r, pl.ds(u * 16, 16)]
                m = (dv >= dlo) & (dv < dhi)
                mi = jnp.where(m, jnp.full((16,), 1, jnp.int32),
                               jnp.full((16,), 0, jnp.int32))
                incl = plsc.cumsum(mi)
                excl = incl - mi
                rsv = jnp.where(sv >= HALF, sv + (HALF_PAD - HALF), sv)
                ldv = dv - dlo
                idx = jnp.where(m, pos3 + excl, TRASH + iota16)
                plsc.store_scatter(fs, [idx], rsv)
                plsc.store_scatter(fd, [idx], ldv)
                plsc.store_scatter(fv, [idx], vv)
                return pos3 + incl[15]
            pos2 = lax.fori_loop(0, 8, g16, pos2)

            do_flush = pos2 >= FLUSH

            @pl.when(do_flush)
            def _flush():
                pltpu.sync_copy(fs.at[pl.ds(0, FLUSH)],
                                osrc.at[pl.ds(pl.multiple_of(base_out + total2, 8), FLUSH)])
                pltpu.sync_copy(fd.at[pl.ds(0, FLUSH)],
                                odst.at[pl.ds(pl.multiple_of(base_out + total2, 8), FLUSH)])
                pltpu.sync_copy(fv.at[pl.ds(0, FLUSH)],
                                oval.at[pl.ds(pl.multiple_of(base_out + total2, 8), FLUSH)])
                ntail = (pos2 - FLUSH + 15) // 16

                def mv(t, _):
                    fs[pl.ds(t * 16, 16)] = fs[pl.ds(FLUSH + t * 16, 16)]
                    fd[pl.ds(t * 16, 16)] = fd[pl.ds(FLUSH + t * 16, 16)]
                    fv[pl.ds(t * 16, 16)] = fv[pl.ds(FLUSH + t * 16, 16)]
                    return 0
                lax.fori_loop(0, ntail, mv, 0)

            pos2 = jnp.where(do_flush, pos2 - FLUSH, pos2)
            total2 = jnp.where(do_flush, total2 + FLUSH, total2)
            return (pos2, total2)
        return lax.fori_loop(0, 8, chunk, (pos, total))

    pos, total = lax.fori_loop(0, IN_CPT // 8, block,
                               (jnp.int32(0), jnp.int32(0)))

    @pl.when(pos > 0)
    def _tail():
        zi = jnp.zeros((16,), jnp.int32)
        zf = jnp.zeros((16,), jnp.float32)
        fs[pl.ds(pos, 16)] = zi
        fd[pl.ds(pos, 16)] = zi
        fv[pl.ds(pos, 16)] = zf
        pos16 = (pos + 15) & ~jnp.int32(15)

        def padk(k, _):
            off = pos16 + k * 16
            fs[pl.ds(off, 16)] = zi
            fd[pl.ds(off, 16)] = zi
            fv[pl.ds(off, 16)] = zf
            return 0
        lax.fori_loop(0, (FLUSH - pos16) // 16, padk, 0)
        pltpu.sync_copy(fs.at[pl.ds(0, FLUSH)],
                        osrc.at[pl.ds(pl.multiple_of(base_out + total, 8), FLUSH)])
        pltpu.sync_copy(fd.at[pl.ds(0, FLUSH)],
                        odst.at[pl.ds(pl.multiple_of(base_out + total, 8), FLUSH)])
        pltpu.sync_copy(fv.at[pl.ds(0, FLUSH)],
                        oval.at[pl.ds(pl.multiple_of(base_out + total, 8), FLUSH)])

    total = jnp.where(pos > 0, total + FLUSH, total)
    cntv[pl.ds(0, 16)] = jnp.broadcast_to(total, (16,)).astype(jnp.int32)
    pltpu.sync_copy(cntv, counts.at[pl.ds(pl.multiple_of(tid * 16, 16), 16)])

    _zero_acc(acc, rows, s)
    plsc.subcore_barrier()
    _edges_phase(ego_tab, osrc, odst, oval, cntv, acc,
                 sbv, dbv, vbv, dx, rows, gsem, ssem, esem, base_out)
    plsc.subcore_barrier()
    _writeback(acc, t1, c, s)


def _body2(tin, osrc, odst, oval, counts,
           tout,
           acc, sbv, dbv, vbv, dx0, dx1, dx2, rows, cntv,
           gs0, gs1, gs2, ss0, ss1, ss2, esem):
    c = lax.axis_index("c")
    s = lax.axis_index("s")
    tid = c * NS + s
    dx = [dx0, dx1, dx2]
    gsem = [gs0, gs1, gs2]
    ssem = [ss0, ss1, ss2]
    pltpu.sync_copy(counts.at[pl.ds(pl.multiple_of(tid * 16, 16), 16)], cntv)
    _zero_acc(acc, rows, s)
    plsc.subcore_barrier()
    _edges_phase(tin, osrc, odst, oval, cntv, acc,
                 sbv, dbv, vbv, dx, rows, gsem, ssem, esem, tid * CAP)
    plsc.subcore_barrier()
    _writeback(acc, tout, c, s)


def _body3(tin, t1, osrc, odst, oval, counts,
           outf,
           acc, sbv, dbv, vbv, dx0, dx1, dx2, rows, cntv,
           gs0, gs1, gs2, ss0, ss1, ss2, esem):
    c = lax.axis_index("c")
    s = lax.axis_index("s")
    tid = c * NS + s
    dx = [dx0, dx1, dx2]
    gsem = [gs0, gs1, gs2]
    ssem = [ss0, ss1, ss2]
    pltpu.sync_copy(counts.at[pl.ds(pl.multiple_of(tid * 16, 16), 16)], cntv)
    _zero_acc(acc, rows, s)
    plsc.subcore_barrier()
    _edges_phase(tin, osrc, odst, oval, cntv, acc,
                 sbv, dbv, vbv, dx, rows, gsem, ssem, esem, tid * CAP)
    plsc.subcore_barrier()

    def fin(z, _):
        gbase = c * HALF_PAD + s * ROWS_PER_TILE + z * ZCH
        abase = s * ROWS_PER_TILE + z * ZCH
        pltpu.sync_copy(t1.at[pl.ds(gbase, ZCH)], rows.at[pl.ds(0, ZCH)])
        pltpu.sync_copy(tin.at[pl.ds(gbase, ZCH)], rows.at[pl.ds(ZCH, ZCH)])
        pltpu.sync_copy(acc.at[pl.ds(abase, ZCH)], rows.at[pl.ds(2 * ZCH, ZCH)])

        def m(i, _):
            for o in (0, 16, 32, 48):
                x = (rows[i, pl.ds(o, 16)] + rows[ZCH + i, pl.ds(o, 16)]
                     + rows[2 * ZCH + i, pl.ds(o, 16)]) * jnp.float32(1.0 / 3.0)
                rows[i, pl.ds(o, 16)] = x
            return 0
        lax.fori_loop(0, ZCH, m, 0)
        pltpu.sync_copy(rows.at[pl.ds(0, ZCH)], outf.at[pl.ds(gbase, ZCH)])
        return 0
    lax.fori_loop(0, NZC, fin, 0)


_MESH = plsc.VectorSubcoreMesh(core_axis_name="c", subcore_axis_name="s")
_PARAMS = pltpu.CompilerParams(use_tc_tiling_on_sc=False,
                               needs_layout_passes=False)

_COMMON_SCRATCH = [
    pltpu.VMEM_SHARED((HALF_PAD, D), jnp.float32),
    pltpu.VMEM((2 * GP * SUB,), jnp.int32),
    pltpu.VMEM((2 * GP * SUB,), jnp.int32),
    pltpu.VMEM((2 * GP * SUB,), jnp.float32),
    pltpu.VMEM((SUB,), jnp.int32),
    pltpu.VMEM((SUB,), jnp.int32),
    pltpu.VMEM((SUB,), jnp.int32),
    pltpu.VMEM((NSL * SUB, D), jnp.float32),
    pltpu.VMEM((16,), jnp.int32),
] + [pltpu.SemaphoreType.DMA] * 7

_run1 = pl.kernel(
    _body1,
    out_type=(jax.ShapeDtypeStruct((NT, D), jnp.float32),
              jax.ShapeDtypeStruct((OSZ,), jnp.int32),
              jax.ShapeDtypeStruct((OSZ,), jnp.int32),
              jax.ShapeDtypeStruct((OSZ,), jnp.float32),
              jax.ShapeDtypeStruct((NC * NS * 16,), jnp.int32)),
    mesh=_MESH,
    compiler_params=_PARAMS,
    scratch_types=[
        pltpu.VMEM_SHARED((HALF_PAD, D), jnp.float32),
        pltpu.VMEM((8, 128), jnp.int32),
        pltpu.VMEM((8, 128), jnp.int32),
        pltpu.VMEM((8, 128), jnp.float32),
        pltpu.VMEM((FIFO,), jnp.int32),
        pltpu.VMEM((FIFO,), jnp.int32),
        pltpu.VMEM((FIFO,), jnp.float32),
        pltpu.VMEM((2 * GP * SUB,), jnp.int32),
        pltpu.VMEM((2 * GP * SUB,), jnp.int32),
        pltpu.VMEM((2 * GP * SUB,), jnp.float32),
        pltpu.VMEM((SUB,), jnp.int32),
        pltpu.VMEM((SUB,), jnp.int32),
        pltpu.VMEM((SUB,), jnp.int32),
        pltpu.VMEM((NSL * SUB, D), jnp.float32),
        pltpu.VMEM((16,), jnp.int32),
    ] + [pltpu.SemaphoreType.DMA] * 7,
)

_run2 = pl.kernel(
    _body2,
    out_type=jax.ShapeDtypeStruct((NT, D), jnp.float32),
    mesh=_MESH,
    compiler_params=_PARAMS,
    scratch_types=list(_COMMON_SCRATCH),
)

_run3 = pl.kernel(
    _body3,
    out_type=jax.ShapeDtypeStruct((NT, D), jnp.float32),
    mesh=_MESH,
    compiler_params=_PARAMS,
    scratch_types=list(_COMMON_SCRATCH),
)


def kernel(user_emb, item_emb, edge_val, edge_src, edge_dst):
    zpad = jnp.zeros((HALF_PAD - HALF, D), jnp.float32)
    ego_tab = jnp.concatenate([user_emb, zpad, item_emb, zpad], axis=0)
    e = edge_src.shape[0]
    pad = E_PAD - e
    srcr = jnp.pad(edge_src.astype(jnp.int32), (0, pad)).reshape(E_PAD // IN_W, IN_W)
    dstr = jnp.pad(edge_dst.astype(jnp.int32), (0, pad), constant_values=-1
                   ).reshape(E_PAD // IN_W, IN_W)
    valr = jnp.pad(edge_val, (0, pad)).reshape(E_PAD // IN_W, IN_W)
    t1, osrc, odst, oval, counts = _run1(ego_tab, srcr, dstr, valr)
    t2 = _run2(t1, osrc, odst, oval, counts)
    outf = _run3(t2, t1, osrc, odst, oval, counts)
    return outf[:N_USERS], outf[HALF_PAD:HALF_PAD + N_ITEMS]


# speedup vs baseline: 1.7043x; 1.7043x over previous
import jax
import jax.numpy as jnp
from jax import lax
from jax.experimental import pallas as pl
from jax.experimental.pallas import tpu as pltpu
from jax.experimental.pallas import tpu_sc as plsc

N_USERS = 25000
N_ITEMS = 25000
N = N_USERS + N_ITEMS
HALF = 25000
HALF_PAD = 25600
NT = 2 * HALF_PAD
D = 64
NC, NS = 2, 16
SUB = 96
GP = 3
NSL = 3
IN_W = 128
IN_CPT = 400
E_PAD = IN_CPT * IN_W * NS
FLUSH = 864
FIFO = FLUSH + SUB + 48
TRASH = FIFO - 16
CAP = 52704
OSZ = NC * NS * CAP
ROWS_PER_TILE = HALF_PAD // NS
ZCH = 64
NZC = ROWS_PER_TILE // ZCH


def _scale_chunk(rows, vbv, vbase, p):
    @plsc.parallel_loop(0, SUB // 16, unroll=2)
    def mg(g8):
        vv = vbv[pl.ds(vbase + g8 * 16, 16)]
        for l in range(16):
            e = p * SUB + g8 * 16 + l
            v = vv[l]
            for o in (0, 16, 32, 48):
                rows[e, pl.ds(o, 16)] = rows[e, pl.ds(o, 16)] * v


def _zero_acc(acc, rows, s):
    def zr(i, _):
        for o in (0, 16, 32, 48):
            rows[i, pl.ds(o, 16)] = jnp.zeros((16,), jnp.float32)
        return 0
    lax.fori_loop(0, ZCH, zr, 0)

    def zc(z, _):
        pltpu.sync_copy(rows.at[pl.ds(0, ZCH)],
                        acc.at[pl.ds(s * ROWS_PER_TILE + z * ZCH, ZCH)])
        return 0
    lax.fori_loop(0, NZC, zc, 0)


def _edges_phase(tab, osrc, odst, oval, cntv, acc,
                 sbv, dbv, vbv, dx, rows, gsem, ssem, esem, base_out):
    total = cntv[pl.ds(0, 16)][0]
    ngroups = total // (GP * SUB)
    base_out = pl.multiple_of(base_out, 8)

    def gather(sl, idx_off):
        pltpu.async_copy(tab.at[sbv.at[pl.ds(idx_off, SUB)]],
                         rows.at[pl.ds(sl * SUB, SUB)], gsem[sl])

    def gather_wait(sl, idx_off):
        pltpu.make_async_copy(tab.at[sbv.at[pl.ds(idx_off, SUB)]],
                              rows.at[pl.ds(sl * SUB, SUB)], gsem[sl]).wait()

    def scatter(sl):
        pltpu.async_copy(rows.at[pl.ds(sl * SUB, SUB)],
                         acc.at[dx[sl]], ssem[sl], add=True)

    def scatter_wait(sl):
        pltpu.make_async_copy(rows.at[pl.ds(sl * SUB, SUB)],
                              acc.at[dx[sl]], ssem[sl]).wait()

    def copy_dst(sl, doff):
        def mk(u, _):
            dx[sl][pl.ds(u * 16, 16)] = dbv[pl.ds(doff + u * 16, 16)]
            return 0
        lax.fori_loop(0, SUB // 16, mk, 0)

    @pl.when(ngroups > 0)
    def _run():
        pltpu.sync_copy(osrc.at[pl.ds(base_out, GP * SUB)],
                        sbv.at[pl.ds(0, GP * SUB)])
        pltpu.sync_copy(odst.at[pl.ds(base_out, GP * SUB)],
                        dbv.at[pl.ds(0, GP * SUB)])
        pltpu.sync_copy(oval.at[pl.ds(base_out, GP * SUB)],
                        vbv.at[pl.ds(0, GP * SUB)])
        for p in (0, 1):
            gather(p, p * SUB)

        def grp(g, _):
            par = g % 2
            npar = (g + 1) % 2
            hb = par * GP * SUB
            nhb = npar * GP * SUB
            nrow = pl.multiple_of(base_out + (g + 1) * GP * SUB, 8)

            @pl.when(g + 1 < ngroups)
            def _stage():
                pltpu.async_copy(osrc.at[pl.ds(nrow, GP * SUB)],
                                 sbv.at[pl.ds(nhb, GP * SUB)], esem)
                pltpu.async_copy(odst.at[pl.ds(nrow, GP * SUB)],
                                 dbv.at[pl.ds(nhb, GP * SUB)], esem)
                pltpu.async_copy(oval.at[pl.ds(nrow, GP * SUB)],
                                 vbv.at[pl.ds(nhb, GP * SUB)], esem)

            for p in range(GP):
                sl2 = (p + 2) % NSL
                if p == 0:
                    @pl.when(g >= 1)
                    def _drain0():
                        scatter_wait(sl2)
                    gather(sl2, hb + 2 * SUB)
                else:
                    @pl.when(g + 1 < ngroups)
                    def _pref():
                        scatter_wait(sl2)
                        if p == 1:
                            pltpu.make_async_copy(
                                osrc.at[pl.ds(nrow, GP * SUB)],
                                sbv.at[pl.ds(nhb, GP * SUB)], esem).wait()
                            pltpu.make_async_copy(
                                odst.at[pl.ds(nrow, GP * SUB)],
                                dbv.at[pl.ds(nhb, GP * SUB)], esem).wait()
                            pltpu.make_async_copy(
                                oval.at[pl.ds(nrow, GP * SUB)],
                                vbv.at[pl.ds(nhb, GP * SUB)], esem).wait()
                        gather(sl2, nhb + (p - 1) * SUB)

                gather_wait(p, hb + p * SUB)
                _scale_chunk(rows, vbv, hb + p * SUB, p)
                copy_dst(p, hb + p * SUB)
                scatter(p)
            return 0
        lax.fori_loop(0, ngroups, grp, 0)
        for sl in range(NSL):
            scatter_wait(sl)


def _writeback(acc, dst_tab, c, s):
    pltpu.sync_copy(
        acc.at[pl.ds(s * ROWS_PER_TILE, ROWS_PER_TILE)],
        dst_tab.at[pl.ds(c * HALF_PAD + s * ROWS_PER_TILE, ROWS_PER_TILE)])


def _body1(ego_tab, srcr, dstr, valr,
           t1, osrc, odst, oval, counts,
           acc, sb, db, vb, fs, fd, fv,
           sbv, dbv, vbv, dx0, dx1, dx2, rows, cntv,
           gs0, gs1, gs2, ss0, ss1, ss2, esem):
    c = lax.axis_index("c")
    s = lax.axis_index("s")
    tid = c * NS + s
    base_out = pl.multiple_of(tid * CAP, 8)
    dx = [dx0, dx1, dx2]
    gsem = [gs0, gs1, gs2]
    ssem = [ss0, ss1, ss2]
    dlo = c * HALF
    dhi = dlo + HALF

    def block(b, carry):
        pos, total = carry
        row0 = s * IN_CPT + b * 8
        pltpu.sync_copy(srcr.at[pl.ds(row0, 8)], sb)
        pltpu.sync_copy(dstr.at[pl.ds(row0, 8)], db)
        pltpu.sync_copy(valr.at[pl.ds(row0, 8)], vb)

        iota16 = lax.iota(jnp.int32, 16)

        def chunk(r, carry2):
            pos2, total2 = carry2

            def g16(u, pos3):
                sv = sb[r, pl.ds(u * 16, 16)]
                dv = db[r, pl.ds(u * 16, 16)]
                vv = vb[r, pl.ds(u * 16, 16)]
                m = (dv >= dlo) & (dv < dhi)
                mi = jnp.where(m, jnp.full((16,), 1, jnp.int32),
                               jnp.full((16,), 0, jnp.int32))
                incl = plsc.cumsum(mi)
                excl = incl - mi
                rsv = jnp.where(sv >= HALF, sv + (HALF_PAD - HALF), sv)
                ldv = dv - dlo
                idx = jnp.where(m, pos3 + excl, TRASH + iota16)
                plsc.store_scatter(fs, [idx], rsv)
                plsc.store_scatter(fd, [idx], ldv)
                plsc.store_scatter(fv, [idx], vv)
                return pos3 + incl[15]
            pos2 = lax.fori_loop(0, 8, g16, pos2)

            do_flush = pos2 >= FLUSH

            @pl.when(do_flush)
            def _flush():
                pltpu.sync_copy(fs.at[pl.ds(0, FLUSH)],
                                osrc.at[pl.ds(pl.multiple_of(base_out + total2, 8), FLUSH)])
                pltpu.sync_copy(fd.at[pl.ds(0, FLUSH)],
                                odst.at[pl.ds(pl.multiple_of(base_out + total2, 8), FLUSH)])
                pltpu.sync_copy(fv.at[pl.ds(0, FLUSH)],
                                oval.at[pl.ds(pl.multiple_of(base_out + total2, 8), FLUSH)])
                ntail = (pos2 - FLUSH + 15) // 16

                def mv(t, _):
                    fs[pl.ds(t * 16, 16)] = fs[pl.ds(FLUSH + t * 16, 16)]
                    fd[pl.ds(t * 16, 16)] = fd[pl.ds(FLUSH + t * 16, 16)]
                    fv[pl.ds(t * 16, 16)] = fv[pl.ds(FLUSH + t * 16, 16)]
                    return 0
                lax.fori_loop(0, ntail, mv, 0)

            pos2 = jnp.where(do_flush, pos2 - FLUSH, pos2)
            total2 = jnp.where(do_flush, total2 + FLUSH, total2)
            return (pos2, total2)
        return lax.fori_loop(0, 8, chunk, (pos, total))

    pos, total = lax.fori_loop(0, IN_CPT // 8, block,
                               (jnp.int32(0), jnp.int32(0)))

    @pl.when(pos > 0)
    def _tail():
        zi = jnp.zeros((16,), jnp.int32)
        zf = jnp.zeros((16,), jnp.float32)
        fs[pl.ds(pos, 16)] = zi
        fd[pl.ds(pos, 16)] = zi
        fv[pl.ds(pos, 16)] = zf
        pos16 = (pos + 15) & ~jnp.int32(15)

        def padk(k, _):
            off = pos16 + k * 16
            fs[pl.ds(off, 16)] = zi
            fd[pl.ds(off, 16)] = zi
            fv[pl.ds(off, 16)] = zf
            return 0
        lax.fori_loop(0, (FLUSH - pos16) // 16, padk, 0)
        pltpu.sync_copy(fs.at[pl.ds(0, FLUSH)],
                        osrc.at[pl.ds(pl.multiple_of(base_out + total, 8), FLUSH)])
        pltpu.sync_copy(fd.at[pl.ds(0, FLUSH)],
                        odst.at[pl.ds(pl.multiple_of(base_out + total, 8), FLUSH)])
        pltpu.sync_copy(fv.at[pl.ds(0, FLUSH)],
                        oval.at[pl.ds(pl.multiple_of(base_out + total, 8), FLUSH)])

    total = jnp.where(pos > 0, total + FLUSH, total)
    cntv[pl.ds(0, 16)] = jnp.broadcast_to(total, (16,)).astype(jnp.int32)
    pltpu.sync_copy(cntv, counts.at[pl.ds(pl.multiple_of(tid * 16, 16), 16)])

    _zero_acc(acc, rows, s)
    plsc.subcore_barrier()
    _edges_phase(ego_tab, osrc, odst, oval, cntv, acc,
                 sbv, dbv, vbv, dx, rows, gsem, ssem, esem, base_out)
    plsc.subcore_barrier()
    _writeback(acc, t1, c, s)


def _body2(tin, osrc, odst, oval, counts,
           tout,
           acc, sbv, dbv, vbv, dx0, dx1, dx2, rows, cntv,
           gs0, gs1, gs2, ss0, ss1, ss2, esem):
    c = lax.axis_index("c")
    s = lax.axis_index("s")
    tid = c * NS + s
    dx = [dx0, dx1, dx2]
    gsem = [gs0, gs1, gs2]
    ssem = [ss0, ss1, ss2]
    pltpu.sync_copy(counts.at[pl.ds(pl.multiple_of(tid * 16, 16), 16)], cntv)
    _zero_acc(acc, rows, s)
    plsc.subcore_barrier()
    _edges_phase(tin, osrc, odst, oval, cntv, acc,
                 sbv, dbv, vbv, dx, rows, gsem, ssem, esem, tid * CAP)
    plsc.subcore_barrier()
    _writeback(acc, tout, c, s)


def _body3(tin, t1, osrc, odst, oval, counts,
           outf,
           acc, sbv, dbv, vbv, dx0, dx1, dx2, rows, cntv,
           gs0, gs1, gs2, ss0, ss1, ss2, esem):
    c = lax.axis_index("c")
    s = lax.axis_index("s")
    tid = c * NS + s
    dx = [dx0, dx1, dx2]
    gsem = [gs0, gs1, gs2]
    ssem = [ss0, ss1, ss2]
    pltpu.sync_copy(counts.at[pl.ds(pl.multiple_of(tid * 16, 16), 16)], cntv)
    _zero_acc(acc, rows, s)
    plsc.subcore_barrier()
    _edges_phase(tin, osrc, odst, oval, cntv, acc,
                 sbv, dbv, vbv, dx, rows, gsem, ssem, esem, tid * CAP)
    plsc.subcore_barrier()

    def fin(z, _):
        gbase = c * HALF_PAD + s * ROWS_PER_TILE + z * ZCH
        abase = s * ROWS_PER_TILE + z * ZCH
        pltpu.sync_copy(t1.at[pl.ds(gbase, ZCH)], rows.at[pl.ds(0, ZCH)])
        pltpu.sync_copy(tin.at[pl.ds(gbase, ZCH)], rows.at[pl.ds(ZCH, ZCH)])
        pltpu.sync_copy(acc.at[pl.ds(abase, ZCH)], rows.at[pl.ds(2 * ZCH, ZCH)])

        def m(i, _):
            for o in (0, 16, 32, 48):
                x = (rows[i, pl.ds(o, 16)] + rows[ZCH + i, pl.ds(o, 16)]
                     + rows[2 * ZCH + i, pl.ds(o, 16)]) * jnp.float32(1.0 / 3.0)
                rows[i, pl.ds(o, 16)] = x
            return 0
        lax.fori_loop(0, ZCH, m, 0)
        pltpu.sync_copy(rows.at[pl.ds(0, ZCH)], outf.at[pl.ds(gbase, ZCH)])
        return 0
    lax.fori_loop(0, NZC, fin, 0)


_MESH = plsc.VectorSubcoreMesh(core_axis_name="c", subcore_axis_name="s")
_PARAMS = pltpu.CompilerParams(use_tc_tiling_on_sc=False,
                               needs_layout_passes=False)

_COMMON_SCRATCH = [
    pltpu.VMEM_SHARED((HALF_PAD, D), jnp.float32),
    pltpu.VMEM((2 * GP * SUB,), jnp.int32),
    pltpu.VMEM((2 * GP * SUB,), jnp.int32),
    pltpu.VMEM((2 * GP * SUB,), jnp.float32),
    pltpu.VMEM((SUB,), jnp.int32),
    pltpu.VMEM((SUB,), jnp.int32),
    pltpu.VMEM((SUB,), jnp.int32),
    pltpu.VMEM((NSL * SUB, D), jnp.float32),
    pltpu.VMEM((16,), jnp.int32),
] + [pltpu.SemaphoreType.DMA] * 7

_run1 = pl.kernel(
    _body1,
    out_type=(jax.ShapeDtypeStruct((NT, D), jnp.float32),
              jax.ShapeDtypeStruct((OSZ,), jnp.int32),
              jax.ShapeDtypeStruct((OSZ,), jnp.int32),
              jax.ShapeDtypeStruct((OSZ,), jnp.float32),
              jax.ShapeDtypeStruct((NC * NS * 16,), jnp.int32)),
    mesh=_MESH,
    compiler_params=_PARAMS,
    scratch_types=[
        pltpu.VMEM_SHARED((HALF_PAD, D), jnp.float32),
        pltpu.VMEM((8, 128), jnp.int32),
        pltpu.VMEM((8, 128), jnp.int32),
        pltpu.VMEM((8, 128), jnp.float32),
        pltpu.VMEM((FIFO,), jnp.int32),
        pltpu.VMEM((FIFO,), jnp.int32),
        pltpu.VMEM((FIFO,), jnp.float32),
        pltpu.VMEM((2 * GP * SUB,), jnp.int32),
        pltpu.VMEM((2 * GP * SUB,), jnp.int32),
        pltpu.VMEM((2 * GP * SUB,), jnp.float32),
        pltpu.VMEM((SUB,), jnp.int32),
        pltpu.VMEM((SUB,), jnp.int32),
        pltpu.VMEM((SUB,), jnp.int32),
        pltpu.VMEM((NSL * SUB, D), jnp.float32),
        pltpu.VMEM((16,), jnp.int32),
    ] + [pltpu.SemaphoreType.DMA] * 7,
)

_run2 = pl.kernel(
    _body2,
    out_type=jax.ShapeDtypeStruct((NT, D), jnp.float32),
    mesh=_MESH,
    compiler_params=_PARAMS,
    scratch_types=list(_COMMON_SCRATCH),
)

_run3 = pl.kernel(
    _body3,
    out_type=jax.ShapeDtypeStruct((NT, D), jnp.float32),
    mesh=_MESH,
    compiler_params=_PARAMS,
    scratch_types=list(_COMMON_SCRATCH),
)


def kernel(user_emb, item_emb, edge_val, edge_src, edge_dst):
    zpad = jnp.zeros((HALF_PAD - HALF, D), jnp.float32)
    ego_tab = jnp.concatenate([user_emb, zpad, item_emb, zpad], axis=0)
    e = edge_src.shape[0]
    pad = E_PAD - e
    srcr = jnp.pad(edge_src.astype(jnp.int32), (0, pad)).reshape(E_PAD // IN_W, IN_W)
    dstr = jnp.pad(edge_dst.astype(jnp.int32), (0, pad), constant_values=-1
                   ).reshape(E_PAD // IN_W, IN_W)
    valr = jnp.pad(edge_val, (0, pad)).reshape(E_PAD // IN_W, IN_W)
    t1, osrc, odst, oval, counts = _run1(ego_tab, srcr, dstr, valr)
    t2 = _run2(t1, osrc, odst, oval, counts)
    outf = _run3(t2, t1, osrc, odst, oval, counts)
    return outf[:N_USERS], outf[HALF_PAD:HALF_PAD + N_ITEMS]


# speedup vs baseline: 1.7112x; 1.0040x over previous
import jax
import jax.numpy as jnp
from jax import lax
from jax.experimental import pallas as pl
from jax.experimental.pallas import tpu as pltpu
from jax.experimental.pallas import tpu_sc as plsc

N_USERS = 25000
N_ITEMS = 25000
N = N_USERS + N_ITEMS
HALF = 25000
HALF_PAD = 25600
NT = 2 * HALF_PAD
D = 64
NC, NS = 2, 16
SUB = 96
GP = 3
NSL = 3
IN_W = 128
IN_CPT = 400
E_PAD = IN_CPT * IN_W * NS
FLUSH = 864
FIFO = FLUSH + SUB + 48
TRASH = FIFO - 16
CAP = 52704
OSZ = NC * NS * CAP
ROWS_PER_TILE = HALF_PAD // NS
ZCH = 64
NZC = ROWS_PER_TILE // ZCH


def _scale_chunk(rows, vbv, vbase, p):
    @plsc.parallel_loop(0, SUB // 16, unroll=2)
    def mg(g8):
        vv = vbv[pl.ds(vbase + g8 * 16, 16)]
        for l in range(16):
            e = p * SUB + g8 * 16 + l
            v = vv[l]
            for o in (0, 16, 32, 48):
                rows[e, pl.ds(o, 16)] = rows[e, pl.ds(o, 16)] * v


def _zero_acc(acc, rows, s):
    @plsc.parallel_loop(0, ZCH, unroll=2)
    def zr(i):
        for o in (0, 16, 32, 48):
            rows[i, pl.ds(o, 16)] = jnp.zeros((16,), jnp.float32)

    def zc(z, _):
        pltpu.sync_copy(rows.at[pl.ds(0, ZCH)],
                        acc.at[pl.ds(s * ROWS_PER_TILE + z * ZCH, ZCH)])
        return 0
    lax.fori_loop(0, NZC, zc, 0)


def _edges_phase(tab, osrc, odst, oval, cntv, acc,
                 sbv, dbv, vbv, dx, rows, gsem, ssem, esem, base_out):
    total = cntv[pl.ds(0, 16)][0]
    ngroups = total // (GP * SUB)
    base_out = pl.multiple_of(base_out, 8)

    def gather(sl, idx_off):
        pltpu.async_copy(tab.at[sbv.at[pl.ds(idx_off, SUB)]],
                         rows.at[pl.ds(sl * SUB, SUB)], gsem[sl])

    def gather_wait(sl, idx_off):
        pltpu.make_async_copy(tab.at[sbv.at[pl.ds(idx_off, SUB)]],
                              rows.at[pl.ds(sl * SUB, SUB)], gsem[sl]).wait()

    def scatter(sl):
        pltpu.async_copy(rows.at[pl.ds(sl * SUB, SUB)],
                         acc.at[dx[sl]], ssem[sl], add=True)

    def scatter_wait(sl):
        pltpu.make_async_copy(rows.at[pl.ds(sl * SUB, SUB)],
                              acc.at[dx[sl]], ssem[sl]).wait()

    def copy_dst(sl, doff):
        @plsc.parallel_loop(0, SUB // 16, unroll=2)
        def mk(u):
            dx[sl][pl.ds(u * 16, 16)] = dbv[pl.ds(doff + u * 16, 16)]

    @pl.when(ngroups > 0)
    def _run():
        pltpu.sync_copy(osrc.at[pl.ds(base_out, GP * SUB)],
                        sbv.at[pl.ds(0, GP * SUB)])
        pltpu.sync_copy(odst.at[pl.ds(base_out, GP * SUB)],
                        dbv.at[pl.ds(0, GP * SUB)])
        pltpu.sync_copy(oval.at[pl.ds(base_out, GP * SUB)],
                        vbv.at[pl.ds(0, GP * SUB)])
        for p in (0, 1):
            gather(p, p * SUB)

        def grp(g, _):
            par = g % 2
            npar = (g + 1) % 2
            hb = par * GP * SUB
            nhb = npar * GP * SUB
            nrow = pl.multiple_of(base_out + (g + 1) * GP * SUB, 8)

            @pl.when(g + 1 < ngroups)
            def _stage():
                pltpu.async_copy(osrc.at[pl.ds(nrow, GP * SUB)],
                                 sbv.at[pl.ds(nhb, GP * SUB)], esem)
                pltpu.async_copy(odst.at[pl.ds(nrow, GP * SUB)],
                                 dbv.at[pl.ds(nhb, GP * SUB)], esem)
                pltpu.async_copy(oval.at[pl.ds(nrow, GP * SUB)],
                                 vbv.at[pl.ds(nhb, GP * SUB)], esem)

            for p in range(GP):
                sl2 = (p + 2) % NSL
                if p == 0:
                    @pl.when(g >= 1)
                    def _drain0():
                        scatter_wait(sl2)
                    gather(sl2, hb + 2 * SUB)
                else:
                    @pl.when(g + 1 < ngroups)
                    def _pref():
                        scatter_wait(sl2)
                        if p == 1:
                            pltpu.make_async_copy(
                                osrc.at[pl.ds(nrow, GP * SUB)],
                                sbv.at[pl.ds(nhb, GP * SUB)], esem).wait()
                            pltpu.make_async_copy(
                                odst.at[pl.ds(nrow, GP * SUB)],
                                dbv.at[pl.ds(nhb, GP * SUB)], esem).wait()
                            pltpu.make_async_copy(
                                oval.at[pl.ds(nrow, GP * SUB)],
                                vbv.at[pl.ds(nhb, GP * SUB)], esem).wait()
                        gather(sl2, nhb + (p - 1) * SUB)

                gather_wait(p, hb + p * SUB)
                _scale_chunk(rows, vbv, hb + p * SUB, p)
                copy_dst(p, hb + p * SUB)
                scatter(p)
            return 0
        lax.fori_loop(0, ngroups, grp, 0)
        for sl in range(NSL):
            scatter_wait(sl)


def _writeback(acc, dst_tab, c, s):
    pltpu.sync_copy(
        acc.at[pl.ds(s * ROWS_PER_TILE, ROWS_PER_TILE)],
        dst_tab.at[pl.ds(c * HALF_PAD + s * ROWS_PER_TILE, ROWS_PER_TILE)])


def _body1(ego_tab, srcr, dstr, valr,
           t1, osrc, odst, oval, counts,
           acc, sb, db, vb, fs, fd, fv,
           sbv, dbv, vbv, dx0, dx1, dx2, rows, cntv,
           gs0, gs1, gs2, ss0, ss1, ss2, esem):
    c = lax.axis_index("c")
    s = lax.axis_index("s")
    tid = c * NS + s
    base_out = pl.multiple_of(tid * CAP, 8)
    dx = [dx0, dx1, dx2]
    gsem = [gs0, gs1, gs2]
    ssem = [ss0, ss1, ss2]
    dlo = c * HALF
    dhi = dlo + HALF

    def block(b, carry):
        pos, total = carry
        row0 = s * IN_CPT + b * 8
        pltpu.sync_copy(srcr.at[pl.ds(row0, 8)], sb)
        pltpu.sync_copy(dstr.at[pl.ds(row0, 8)], db)
        pltpu.sync_copy(valr.at[pl.ds(row0, 8)], vb)

        iota16 = lax.iota(jnp.int32, 16)

        def chunk(r, carry2):
            pos2, total2 = carry2

            def g16(u, pos3):
                sv = sb[r, pl.ds(u * 16, 16)]
                dv = db[r, pl.ds(u * 16, 16)]
                vv = vb[r, pl.ds(u * 16, 16)]
                m = (dv >= dlo) & (dv < dhi)
                mi = jnp.where(m, jnp.full((16,), 1, jnp.int32),
                               jnp.full((16,), 0, jnp.int32))
                incl = plsc.cumsum(mi)
                excl = incl - mi
                rsv = jnp.where(sv >= HALF, sv + (HALF_PAD - HALF), sv)
                ldv = dv - dlo
                idx = jnp.where(m, pos3 + excl, TRASH + iota16)
                plsc.store_scatter(fs, [idx], rsv)
                plsc.store_scatter(fd, [idx], ldv)
                plsc.store_scatter(fv, [idx], vv)
                return pos3 + incl[15]
            pos2 = lax.fori_loop(0, 8, g16, pos2)

            do_flush = pos2 >= FLUSH

            @pl.when(do_flush)
            def _flush():
                pltpu.sync_copy(fs.at[pl.ds(0, FLUSH)],
                                osrc.at[pl.ds(pl.multiple_of(base_out + total2, 8), FLUSH)])
                pltpu.sync_copy(fd.at[pl.ds(0, FLUSH)],
                                odst.at[pl.ds(pl.multiple_of(base_out + total2, 8), FLUSH)])
                pltpu.sync_copy(fv.at[pl.ds(0, FLUSH)],
                                oval.at[pl.ds(pl.multiple_of(base_out + total2, 8), FLUSH)])
                ntail = (pos2 - FLUSH + 15) // 16

                def mv(t, _):
                    fs[pl.ds(t * 16, 16)] = fs[pl.ds(FLUSH + t * 16, 16)]
                    fd[pl.ds(t * 16, 16)] = fd[pl.ds(FLUSH + t * 16, 16)]
                    fv[pl.ds(t * 16, 16)] = fv[pl.ds(FLUSH + t * 16, 16)]
                    return 0
                lax.fori_loop(0, ntail, mv, 0)

            pos2 = jnp.where(do_flush, pos2 - FLUSH, pos2)
            total2 = jnp.where(do_flush, total2 + FLUSH, total2)
            return (pos2, total2)
        return lax.fori_loop(0, 8, chunk, (pos, total))

    pos, total = lax.fori_loop(0, IN_CPT // 8, block,
                               (jnp.int32(0), jnp.int32(0)))

    @pl.when(pos > 0)
    def _tail():
        zi = jnp.zeros((16,), jnp.int32)
        zf = jnp.zeros((16,), jnp.float32)
        fs[pl.ds(pos, 16)] = zi
        fd[pl.ds(pos, 16)] = zi
        fv[pl.ds(pos, 16)] = zf
        pos16 = (pos + 15) & ~jnp.int32(15)

        def padk(k, _):
            off = pos16 + k * 16
            fs[pl.ds(off, 16)] = zi
            fd[pl.ds(off, 16)] = zi
            fv[pl.ds(off, 16)] = zf
            return 0
        lax.fori_loop(0, (FLUSH - pos16) // 16, padk, 0)
        pltpu.sync_copy(fs.at[pl.ds(0, FLUSH)],
                        osrc.at[pl.ds(pl.multiple_of(base_out + total, 8), FLUSH)])
        pltpu.sync_copy(fd.at[pl.ds(0, FLUSH)],
                        odst.at[pl.ds(pl.multiple_of(base_out + total, 8), FLUSH)])
        pltpu.sync_copy(fv.at[pl.ds(0, FLUSH)],
                        oval.at[pl.ds(pl.multiple_of(base_out + total, 8), FLUSH)])

    total = jnp.where(pos > 0, total + FLUSH, total)
    cntv[pl.ds(0, 16)] = jnp.broadcast_to(total, (16,)).astype(jnp.int32)
    pltpu.sync_copy(cntv, counts.at[pl.ds(pl.multiple_of(tid * 16, 16), 16)])

    _zero_acc(acc, rows, s)
    plsc.subcore_barrier()
    _edges_phase(ego_tab, osrc, odst, oval, cntv, acc,
                 sbv, dbv, vbv, dx, rows, gsem, ssem, esem, base_out)
    plsc.subcore_barrier()
    _writeback(acc, t1, c, s)


def _body2(tin, osrc, odst, oval, counts,
           tout,
           acc, sbv, dbv, vbv, dx0, dx1, dx2, rows, cntv,
           gs0, gs1, gs2, ss0, ss1, ss2, esem):
    c = lax.axis_index("c")
    s = lax.axis_index("s")
    tid = c * NS + s
    dx = [dx0, dx1, dx2]
    gsem = [gs0, gs1, gs2]
    ssem = [ss0, ss1, ss2]
    pltpu.sync_copy(counts.at[pl.ds(pl.multiple_of(tid * 16, 16), 16)], cntv)
    _zero_acc(acc, rows, s)
    plsc.subcore_barrier()
    _edges_phase(tin, osrc, odst, oval, cntv, acc,
                 sbv, dbv, vbv, dx, rows, gsem, ssem, esem, tid * CAP)
    plsc.subcore_barrier()
    _writeback(acc, tout, c, s)


def _body3(tin, t1, osrc, odst, oval, counts,
           outf,
           acc, sbv, dbv, vbv, dx0, dx1, dx2, rows, cntv,
           gs0, gs1, gs2, ss0, ss1, ss2, esem):
    c = lax.axis_index("c")
    s = lax.axis_index("s")
    tid = c * NS + s
    dx = [dx0, dx1, dx2]
    gsem = [gs0, gs1, gs2]
    ssem = [ss0, ss1, ss2]
    pltpu.sync_copy(counts.at[pl.ds(pl.multiple_of(tid * 16, 16), 16)], cntv)
    _zero_acc(acc, rows, s)
    plsc.subcore_barrier()
    _edges_phase(tin, osrc, odst, oval, cntv, acc,
                 sbv, dbv, vbv, dx, rows, gsem, ssem, esem, tid * CAP)
    plsc.subcore_barrier()

    def fin(z, _):
        gbase = c * HALF_PAD + s * ROWS_PER_TILE + z * ZCH
        abase = s * ROWS_PER_TILE + z * ZCH
        pltpu.sync_copy(t1.at[pl.ds(gbase, ZCH)], rows.at[pl.ds(0, ZCH)])
        pltpu.sync_copy(tin.at[pl.ds(gbase, ZCH)], rows.at[pl.ds(ZCH, ZCH)])
        pltpu.sync_copy(acc.at[pl.ds(abase, ZCH)], rows.at[pl.ds(2 * ZCH, ZCH)])

        @plsc.parallel_loop(0, ZCH, unroll=2)
        def m(i):
            for o in (0, 16, 32, 48):
                x = (rows[i, pl.ds(o, 16)] + rows[ZCH + i, pl.ds(o, 16)]
                     + rows[2 * ZCH + i, pl.ds(o, 16)]) * jnp.float32(1.0 / 3.0)
                rows[i, pl.ds(o, 16)] = x
        pltpu.sync_copy(rows.at[pl.ds(0, ZCH)], outf.at[pl.ds(gbase, ZCH)])
        return 0
    lax.fori_loop(0, NZC, fin, 0)


_MESH = plsc.VectorSubcoreMesh(core_axis_name="c", subcore_axis_name="s")
_PARAMS = pltpu.CompilerParams(use_tc_tiling_on_sc=False,
                               needs_layout_passes=False)

_COMMON_SCRATCH = [
    pltpu.VMEM_SHARED((HALF_PAD, D), jnp.float32),
    pltpu.VMEM((2 * GP * SUB,), jnp.int32),
    pltpu.VMEM((2 * GP * SUB,), jnp.int32),
    pltpu.VMEM((2 * GP * SUB,), jnp.float32),
    pltpu.VMEM((SUB,), jnp.int32),
    pltpu.VMEM((SUB,), jnp.int32),
    pltpu.VMEM((SUB,), jnp.int32),
    pltpu.VMEM((NSL * SUB, D), jnp.float32),
    pltpu.VMEM((16,), jnp.int32),
] + [pltpu.SemaphoreType.DMA] * 7

_run1 = pl.kernel(
    _body1,
    out_type=(jax.ShapeDtypeStruct((NT, D), jnp.float32),
              jax.ShapeDtypeStruct((OSZ,), jnp.int32),
              jax.ShapeDtypeStruct((OSZ,), jnp.int32),
              jax.ShapeDtypeStruct((OSZ,), jnp.float32),
              jax.ShapeDtypeStruct((NC * NS * 16,), jnp.int32)),
    mesh=_MESH,
    compiler_params=_PARAMS,
    scratch_types=[
        pltpu.VMEM_SHARED((HALF_PAD, D), jnp.float32),
        pltpu.VMEM((8, 128), jnp.int32),
        pltpu.VMEM((8, 128), jnp.int32),
        pltpu.VMEM((8, 128), jnp.float32),
        pltpu.VMEM((FIFO,), jnp.int32),
        pltpu.VMEM((FIFO,), jnp.int32),
        pltpu.VMEM((FIFO,), jnp.float32),
        pltpu.VMEM((2 * GP * SUB,), jnp.int32),
        pltpu.VMEM((2 * GP * SUB,), jnp.int32),
        pltpu.VMEM((2 * GP * SUB,), jnp.float32),
        pltpu.VMEM((SUB,), jnp.int32),
        pltpu.VMEM((SUB,), jnp.int32),
        pltpu.VMEM((SUB,), jnp.int32),
        pltpu.VMEM((NSL * SUB, D), jnp.float32),
        pltpu.VMEM((16,), jnp.int32),
    ] + [pltpu.SemaphoreType.DMA] * 7,
)

_run2 = pl.kernel(
    _body2,
    out_type=jax.ShapeDtypeStruct((NT, D), jnp.float32),
    mesh=_MESH,
    compiler_params=_PARAMS,
    scratch_types=list(_COMMON_SCRATCH),
)

_run3 = pl.kernel(
    _body3,
    out_type=jax.ShapeDtypeStruct((NT, D), jnp.float32),
    mesh=_MESH,
    compiler_params=_PARAMS,
    scratch_types=list(_COMMON_SCRATCH),
)


def kernel(user_emb, item_emb, edge_val, edge_src, edge_dst):
    zpad = jnp.zeros((HALF_PAD - HALF, D), jnp.float32)
    ego_tab = jnp.concatenate([user_emb, zpad, item_emb, zpad], axis=0)
    e = edge_src.shape[0]
    pad = E_PAD - e
    srcr = jnp.pad(edge_src.astype(jnp.int32), (0, pad)).reshape(E_PAD // IN_W, IN_W)
    dstr = jnp.pad(edge_dst.astype(jnp.int32), (0, pad), constant_values=-1
                   ).reshape(E_PAD // IN_W, IN_W)
    valr = jnp.pad(edge_val, (0, pad)).reshape(E_PAD // IN_W, IN_W)
    t1, osrc, odst, oval, counts = _run1(ego_tab, srcr, dstr, valr)
    t2 = _run2(t1, osrc, odst, oval, counts)
    outf = _run3(t2, t1, osrc, odst, oval, counts)
    return outf[:N_USERS], outf[HALF_PAD:HALF_PAD + N_ITEMS]


# speedup vs baseline: 1.7511x; 1.0233x over previous
import jax
import jax.numpy as jnp
from jax import lax
from jax.experimental import pallas as pl
from jax.experimental.pallas import tpu as pltpu
from jax.experimental.pallas import tpu_sc as plsc

N_USERS = 25000
N_ITEMS = 25000
N = N_USERS + N_ITEMS
HALF = 25000
HALF_PAD = 25600
NT = 2 * HALF_PAD
D = 64
NC, NS = 2, 16
SUB = 96
GP = 3
NSL = 3
IN_W = 128
IN_CPT = 400
E_PAD = IN_CPT * IN_W * NS
FLUSH = 864
FIFO = FLUSH + SUB + 48
TRASH = FIFO - 16
CAP = 52704
OSZ = NC * NS * CAP
ROWS_PER_TILE = HALF_PAD // NS
ZCH = 64
NZC = ROWS_PER_TILE // ZCH


def _scale_chunk(rows, vbv, vbase, p):
    @plsc.parallel_loop(0, SUB // 16, unroll=2)
    def mg(g8):
        vv = vbv[pl.ds(vbase + g8 * 16, 16)]
        for l in range(16):
            e = p * SUB + g8 * 16 + l
            v = vv[l]
            for o in (0, 16, 32, 48):
                rows[e, pl.ds(o, 16)] = rows[e, pl.ds(o, 16)] * v


def _zero_acc(acc, rows, s):
    @plsc.parallel_loop(0, ZCH, unroll=2)
    def zr(i):
        for o in (0, 16, 32, 48):
            rows[i, pl.ds(o, 16)] = jnp.zeros((16,), jnp.float32)

    def zc(z, _):
        pltpu.sync_copy(rows.at[pl.ds(0, ZCH)],
                        acc.at[pl.ds(s * ROWS_PER_TILE + z * ZCH, ZCH)])
        return 0
    lax.fori_loop(0, NZC, zc, 0)


def _edges_phase(tab, osrc, odst, oval, cntv, acc,
                 sbv, dbv, vbv, dx, rows, gsem, ssem, esem, base_out):
    total = cntv[pl.ds(0, 16)][0]
    ngroups = total // (GP * SUB)
    base_out = pl.multiple_of(base_out, 8)

    def gather(sl, idx_off):
        pltpu.async_copy(tab.at[sbv.at[pl.ds(idx_off, SUB)]],
                         rows.at[pl.ds(sl * SUB, SUB)], gsem[sl])

    def gather_wait(sl, idx_off):
        pltpu.make_async_copy(tab.at[sbv.at[pl.ds(idx_off, SUB)]],
                              rows.at[pl.ds(sl * SUB, SUB)], gsem[sl]).wait()

    def scatter(sl):
        pltpu.async_copy(rows.at[pl.ds(sl * SUB, SUB)],
                         acc.at[dx[sl]], ssem[sl], add=True)

    def scatter_wait(sl):
        pltpu.make_async_copy(rows.at[pl.ds(sl * SUB, SUB)],
                              acc.at[dx[sl]], ssem[sl]).wait()

    def copy_dst(sl, doff):
        @plsc.parallel_loop(0, SUB // 16, unroll=2)
        def mk(u):
            dx[sl][pl.ds(u * 16, 16)] = dbv[pl.ds(doff + u * 16, 16)]

    @pl.when(ngroups > 0)
    def _run():
        pltpu.sync_copy(osrc.at[pl.ds(base_out, GP * SUB)],
                        sbv.at[pl.ds(0, GP * SUB)])
        pltpu.sync_copy(odst.at[pl.ds(base_out, GP * SUB)],
                        dbv.at[pl.ds(0, GP * SUB)])
        pltpu.sync_copy(oval.at[pl.ds(base_out, GP * SUB)],
                        vbv.at[pl.ds(0, GP * SUB)])
        for p in (0, 1):
            gather(p, p * SUB)

        def grp(g, _):
            par = g % 2
            npar = (g + 1) % 2
            hb = par * GP * SUB
            nhb = npar * GP * SUB
            nrow = pl.multiple_of(base_out + (g + 1) * GP * SUB, 8)

            @pl.when(g + 1 < ngroups)
            def _stage():
                pltpu.async_copy(osrc.at[pl.ds(nrow, GP * SUB)],
                                 sbv.at[pl.ds(nhb, GP * SUB)], esem)
                pltpu.async_copy(odst.at[pl.ds(nrow, GP * SUB)],
                                 dbv.at[pl.ds(nhb, GP * SUB)], esem)
                pltpu.async_copy(oval.at[pl.ds(nrow, GP * SUB)],
                                 vbv.at[pl.ds(nhb, GP * SUB)], esem)

            for p in range(GP):
                sl2 = (p + 2) % NSL
                if p == 0:
                    @pl.when(g >= 1)
                    def _drain0():
                        scatter_wait(sl2)
                    gather(sl2, hb + 2 * SUB)
                else:
                    @pl.when(g + 1 < ngroups)
                    def _pref():
                        scatter_wait(sl2)
                        if p == 1:
                            pltpu.make_async_copy(
                                osrc.at[pl.ds(nrow, GP * SUB)],
                                sbv.at[pl.ds(nhb, GP * SUB)], esem).wait()
                            pltpu.make_async_copy(
                                odst.at[pl.ds(nrow, GP * SUB)],
                                dbv.at[pl.ds(nhb, GP * SUB)], esem).wait()
                            pltpu.make_async_copy(
                                oval.at[pl.ds(nrow, GP * SUB)],
                                vbv.at[pl.ds(nhb, GP * SUB)], esem).wait()
                        gather(sl2, nhb + (p - 1) * SUB)

                gather_wait(p, hb + p * SUB)
                _scale_chunk(rows, vbv, hb + p * SUB, p)
                copy_dst(p, hb + p * SUB)
                scatter(p)
            return 0
        lax.fori_loop(0, ngroups, grp, 0)
        for sl in range(NSL):
            scatter_wait(sl)


def _writeback(acc, dst_tab, c, s):
    pltpu.sync_copy(
        acc.at[pl.ds(s * ROWS_PER_TILE, ROWS_PER_TILE)],
        dst_tab.at[pl.ds(c * HALF_PAD + s * ROWS_PER_TILE, ROWS_PER_TILE)])


def _body1(ego_tab, srcr, dstr, valr,
           t1, osrc, odst, oval, counts,
           acc, sb, db, vb, fs, fd, fv,
           sbv, dbv, vbv, dx0, dx1, dx2, rows, cntv,
           gs0, gs1, gs2, ss0, ss1, ss2, esem):
    c = lax.axis_index("c")
    s = lax.axis_index("s")
    tid = c * NS + s
    base_out = pl.multiple_of(tid * CAP, 8)
    dx = [dx0, dx1, dx2]
    gsem = [gs0, gs1, gs2]
    ssem = [ss0, ss1, ss2]
    dlo = c * HALF
    dhi = dlo + HALF

    def block(b, carry):
        pos, total = carry
        row0 = s * IN_CPT + b * 8
        pltpu.sync_copy(srcr.at[pl.ds(row0, 8)], sb)
        pltpu.sync_copy(dstr.at[pl.ds(row0, 8)], db)
        pltpu.sync_copy(valr.at[pl.ds(row0, 8)], vb)

        iota16 = lax.iota(jnp.int32, 16)

        def chunk(r, carry2):
            pos2, total2 = carry2

            @plsc.parallel_loop(0, 8, unroll=2, carry=pos2)
            def g16(u, pos3):
                sv = sb[r, pl.ds(u * 16, 16)]
                dv = db[r, pl.ds(u * 16, 16)]
                vv = vb[r, pl.ds(u * 16, 16)]
                m = (dv >= dlo) & (dv < dhi)
                mi = jnp.where(m, jnp.full((16,), 1, jnp.int32),
                               jnp.full((16,), 0, jnp.int32))
                incl = plsc.cumsum(mi)
                excl = incl - mi
                rsv = jnp.where(sv >= HALF, sv + (HALF_PAD - HALF), sv)
                ldv = dv - dlo
                idx = jnp.where(m, pos3 + excl, TRASH + iota16)
                plsc.store_scatter(fs, [idx], rsv)
                plsc.store_scatter(fd, [idx], ldv)
                plsc.store_scatter(fv, [idx], vv)
                return pos3 + incl[15]
            pos2 = g16

            do_flush = pos2 >= FLUSH

            @pl.when(do_flush)
            def _flush():
                pltpu.sync_copy(fs.at[pl.ds(0, FLUSH)],
                                osrc.at[pl.ds(pl.multiple_of(base_out + total2, 8), FLUSH)])
                pltpu.sync_copy(fd.at[pl.ds(0, FLUSH)],
                                odst.at[pl.ds(pl.multiple_of(base_out + total2, 8), FLUSH)])
                pltpu.sync_copy(fv.at[pl.ds(0, FLUSH)],
                                oval.at[pl.ds(pl.multiple_of(base_out + total2, 8), FLUSH)])
                ntail = (pos2 - FLUSH + 15) // 16

                def mv(t, _):
                    fs[pl.ds(t * 16, 16)] = fs[pl.ds(FLUSH + t * 16, 16)]
                    fd[pl.ds(t * 16, 16)] = fd[pl.ds(FLUSH + t * 16, 16)]
                    fv[pl.ds(t * 16, 16)] = fv[pl.ds(FLUSH + t * 16, 16)]
                    return 0
                lax.fori_loop(0, ntail, mv, 0)

            pos2 = jnp.where(do_flush, pos2 - FLUSH, pos2)
            total2 = jnp.where(do_flush, total2 + FLUSH, total2)
            return (pos2, total2)
        return lax.fori_loop(0, 8, chunk, (pos, total))

    pos, total = lax.fori_loop(0, IN_CPT // 8, block,
                               (jnp.int32(0), jnp.int32(0)))

    @pl.when(pos > 0)
    def _tail():
        zi = jnp.zeros((16,), jnp.int32)
        zf = jnp.zeros((16,), jnp.float32)
        fs[pl.ds(pos, 16)] = zi
        fd[pl.ds(pos, 16)] = zi
        fv[pl.ds(pos, 16)] = zf
        pos16 = (pos + 15) & ~jnp.int32(15)

        def padk(k, _):
            off = pos16 + k * 16
            fs[pl.ds(off, 16)] = zi
            fd[pl.ds(off, 16)] = zi
            fv[pl.ds(off, 16)] = zf
            return 0
        lax.fori_loop(0, (FLUSH - pos16) // 16, padk, 0)
        pltpu.sync_copy(fs.at[pl.ds(0, FLUSH)],
                        osrc.at[pl.ds(pl.multiple_of(base_out + total, 8), FLUSH)])
        pltpu.sync_copy(fd.at[pl.ds(0, FLUSH)],
                        odst.at[pl.ds(pl.multiple_of(base_out + total, 8), FLUSH)])
        pltpu.sync_copy(fv.at[pl.ds(0, FLUSH)],
                        oval.at[pl.ds(pl.multiple_of(base_out + total, 8), FLUSH)])

    total = jnp.where(pos > 0, total + FLUSH, total)
    cntv[pl.ds(0, 16)] = jnp.broadcast_to(total, (16,)).astype(jnp.int32)
    pltpu.sync_copy(cntv, counts.at[pl.ds(pl.multiple_of(tid * 16, 16), 16)])

    _zero_acc(acc, rows, s)
    plsc.subcore_barrier()
    _edges_phase(ego_tab, osrc, odst, oval, cntv, acc,
                 sbv, dbv, vbv, dx, rows, gsem, ssem, esem, base_out)
    plsc.subcore_barrier()
    _writeback(acc, t1, c, s)


def _body2(tin, osrc, odst, oval, counts,
           tout,
           acc, sbv, dbv, vbv, dx0, dx1, dx2, rows, cntv,
           gs0, gs1, gs2, ss0, ss1, ss2, esem):
    c = lax.axis_index("c")
    s = lax.axis_index("s")
    tid = c * NS + s
    dx = [dx0, dx1, dx2]
    gsem = [gs0, gs1, gs2]
    ssem = [ss0, ss1, ss2]
    pltpu.sync_copy(counts.at[pl.ds(pl.multiple_of(tid * 16, 16), 16)], cntv)
    _zero_acc(acc, rows, s)
    plsc.subcore_barrier()
    _edges_phase(tin, osrc, odst, oval, cntv, acc,
                 sbv, dbv, vbv, dx, rows, gsem, ssem, esem, tid * CAP)
    plsc.subcore_barrier()
    _writeback(acc, tout, c, s)


def _body3(tin, t1, osrc, odst, oval, counts,
           outf,
           acc, sbv, dbv, vbv, dx0, dx1, dx2, rows, cntv,
           gs0, gs1, gs2, ss0, ss1, ss2, esem):
    c = lax.axis_index("c")
    s = lax.axis_index("s")
    tid = c * NS + s
    dx = [dx0, dx1, dx2]
    gsem = [gs0, gs1, gs2]
    ssem = [ss0, ss1, ss2]
    pltpu.sync_copy(counts.at[pl.ds(pl.multiple_of(tid * 16, 16), 16)], cntv)
    _zero_acc(acc, rows, s)
    plsc.subcore_barrier()
    _edges_phase(tin, osrc, odst, oval, cntv, acc,
                 sbv, dbv, vbv, dx, rows, gsem, ssem, esem, tid * CAP)
    plsc.subcore_barrier()

    def fin(z, _):
        gbase = c * HALF_PAD + s * ROWS_PER_TILE + z * ZCH
        abase = s * ROWS_PER_TILE + z * ZCH
        pltpu.sync_copy(t1.at[pl.ds(gbase, ZCH)], rows.at[pl.ds(0, ZCH)])
        pltpu.sync_copy(tin.at[pl.ds(gbase, ZCH)], rows.at[pl.ds(ZCH, ZCH)])
        pltpu.sync_copy(acc.at[pl.ds(abase, ZCH)], rows.at[pl.ds(2 * ZCH, ZCH)])

        @plsc.parallel_loop(0, ZCH, unroll=2)
        def m(i):
            for o in (0, 16, 32, 48):
                x = (rows[i, pl.ds(o, 16)] + rows[ZCH + i, pl.ds(o, 16)]
                     + rows[2 * ZCH + i, pl.ds(o, 16)]) * jnp.float32(1.0 / 3.0)
                rows[i, pl.ds(o, 16)] = x
        pltpu.sync_copy(rows.at[pl.ds(0, ZCH)], outf.at[pl.ds(gbase, ZCH)])
        return 0
    lax.fori_loop(0, NZC, fin, 0)


_MESH = plsc.VectorSubcoreMesh(core_axis_name="c", subcore_axis_name="s")
_PARAMS = pltpu.CompilerParams(use_tc_tiling_on_sc=False,
                               needs_layout_passes=False)

_COMMON_SCRATCH = [
    pltpu.VMEM_SHARED((HALF_PAD, D), jnp.float32),
    pltpu.VMEM((2 * GP * SUB,), jnp.int32),
    pltpu.VMEM((2 * GP * SUB,), jnp.int32),
    pltpu.VMEM((2 * GP * SUB,), jnp.float32),
    pltpu.VMEM((SUB,), jnp.int32),
    pltpu.VMEM((SUB,), jnp.int32),
    pltpu.VMEM((SUB,), jnp.int32),
    pltpu.VMEM((NSL * SUB, D), jnp.float32),
    pltpu.VMEM((16,), jnp.int32),
] + [pltpu.SemaphoreType.DMA] * 7

_run1 = pl.kernel(
    _body1,
    out_type=(jax.ShapeDtypeStruct((NT, D), jnp.float32),
              jax.ShapeDtypeStruct((OSZ,), jnp.int32),
              jax.ShapeDtypeStruct((OSZ,), jnp.int32),
              jax.ShapeDtypeStruct((OSZ,), jnp.float32),
              jax.ShapeDtypeStruct((NC * NS * 16,), jnp.int32)),
    mesh=_MESH,
    compiler_params=_PARAMS,
    scratch_types=[
        pltpu.VMEM_SHARED((HALF_PAD, D), jnp.float32),
        pltpu.VMEM((8, 128), jnp.int32),
        pltpu.VMEM((8, 128), jnp.int32),
        pltpu.VMEM((8, 128), jnp.float32),
        pltpu.VMEM((FIFO,), jnp.int32),
        pltpu.VMEM((FIFO,), jnp.int32),
        pltpu.VMEM((FIFO,), jnp.float32),
        pltpu.VMEM((2 * GP * SUB,), jnp.int32),
        pltpu.VMEM((2 * GP * SUB,), jnp.int32),
        pltpu.VMEM((2 * GP * SUB,), jnp.float32),
        pltpu.VMEM((SUB,), jnp.int32),
        pltpu.VMEM((SUB,), jnp.int32),
        pltpu.VMEM((SUB,), jnp.int32),
        pltpu.VMEM((NSL * SUB, D), jnp.float32),
        pltpu.VMEM((16,), jnp.int32),
    ] + [pltpu.SemaphoreType.DMA] * 7,
)

_run2 = pl.kernel(
    _body2,
    out_type=jax.ShapeDtypeStruct((NT, D), jnp.float32),
    mesh=_MESH,
    compiler_params=_PARAMS,
    scratch_types=list(_COMMON_SCRATCH),
)

_run3 = pl.kernel(
    _body3,
    out_type=jax.ShapeDtypeStruct((NT, D), jnp.float32),
    mesh=_MESH,
    compiler_params=_PARAMS,
    scratch_types=list(_COMMON_SCRATCH),
)


def kernel(user_emb, item_emb, edge_val, edge_src, edge_dst):
    zpad = jnp.zeros((HALF_PAD - HALF, D), jnp.float32)
    ego_tab = jnp.concatenate([user_emb, zpad, item_emb, zpad], axis=0)
    e = edge_src.shape[0]
    pad = E_PAD - e
    srcr = jnp.pad(edge_src.astype(jnp.int32), (0, pad)).reshape(E_PAD // IN_W, IN_W)
    dstr = jnp.pad(edge_dst.astype(jnp.int32), (0, pad), constant_values=-1
                   ).reshape(E_PAD // IN_W, IN_W)
    valr = jnp.pad(edge_val, (0, pad)).reshape(E_PAD // IN_W, IN_W)
    t1, osrc, odst, oval, counts = _run1(ego_tab, srcr, dstr, valr)
    t2 = _run2(t1, osrc, odst, oval, counts)
    outf = _run3(t2, t1, osrc, odst, oval, counts)
    return outf[:N_USERS], outf[HALF_PAD:HALF_PAD + N_ITEMS]


# speedup vs baseline: 1.7536x; 1.0014x over previous
import jax
import jax.numpy as jnp
from jax import lax
from jax.experimental import pallas as pl
from jax.experimental.pallas import tpu as pltpu
from jax.experimental.pallas import tpu_sc as plsc

N_USERS = 25000
N_ITEMS = 25000
N = N_USERS + N_ITEMS
HALF = 25000
HALF_PAD = 25600
NT = 2 * HALF_PAD
D = 64
NC, NS = 2, 16
SUB = 96
GP = 3
NSL = 3
IN_W = 128
IN_CPT = 400
E_PAD = IN_CPT * IN_W * NS
FLUSH = 864
FIFO = FLUSH + SUB + 48
TRASH = FIFO - 16
CAP = 52704
OSZ = NC * NS * CAP
ROWS_PER_TILE = HALF_PAD // NS
ZCH = 64
NZC = ROWS_PER_TILE // ZCH


def _scale_chunk(rows, vbv, vbase, p):
    @plsc.parallel_loop(0, SUB // 16, unroll=3)
    def mg(g8):
        vv = vbv[pl.ds(vbase + g8 * 16, 16)]
        for l in range(16):
            e = p * SUB + g8 * 16 + l
            v = vv[l]
            for o in (0, 16, 32, 48):
                rows[e, pl.ds(o, 16)] = rows[e, pl.ds(o, 16)] * v


def _zero_acc(acc, rows, s):
    @plsc.parallel_loop(0, ZCH, unroll=2)
    def zr(i):
        for o in (0, 16, 32, 48):
            rows[i, pl.ds(o, 16)] = jnp.zeros((16,), jnp.float32)

    def zc(z, _):
        pltpu.sync_copy(rows.at[pl.ds(0, ZCH)],
                        acc.at[pl.ds(s * ROWS_PER_TILE + z * ZCH, ZCH)])
        return 0
    lax.fori_loop(0, NZC, zc, 0)


def _edges_phase(tab, osrc, odst, oval, cntv, acc,
                 sbv, dbv, vbv, dx, rows, gsem, ssem, esem, base_out):
    total = cntv[pl.ds(0, 16)][0]
    ngroups = total // (GP * SUB)
    base_out = pl.multiple_of(base_out, 8)

    def gather(sl, idx_off):
        pltpu.async_copy(tab.at[sbv.at[pl.ds(idx_off, SUB)]],
                         rows.at[pl.ds(sl * SUB, SUB)], gsem[sl])

    def gather_wait(sl, idx_off):
        pltpu.make_async_copy(tab.at[sbv.at[pl.ds(idx_off, SUB)]],
                              rows.at[pl.ds(sl * SUB, SUB)], gsem[sl]).wait()

    def scatter(sl):
        pltpu.async_copy(rows.at[pl.ds(sl * SUB, SUB)],
                         acc.at[dx[sl]], ssem[sl], add=True)

    def scatter_wait(sl):
        pltpu.make_async_copy(rows.at[pl.ds(sl * SUB, SUB)],
                              acc.at[dx[sl]], ssem[sl]).wait()

    def copy_dst(sl, doff):
        @plsc.parallel_loop(0, SUB // 16, unroll=2)
        def mk(u):
            dx[sl][pl.ds(u * 16, 16)] = dbv[pl.ds(doff + u * 16, 16)]

    @pl.when(ngroups > 0)
    def _run():
        pltpu.sync_copy(osrc.at[pl.ds(base_out, GP * SUB)],
                        sbv.at[pl.ds(0, GP * SUB)])
        pltpu.sync_copy(odst.at[pl.ds(base_out, GP * SUB)],
                        dbv.at[pl.ds(0, GP * SUB)])
        pltpu.sync_copy(oval.at[pl.ds(base_out, GP * SUB)],
                        vbv.at[pl.ds(0, GP * SUB)])
        for p in (0, 1):
            gather(p, p * SUB)

        def grp(g, _):
            par = g % 2
            npar = (g + 1) % 2
            hb = par * GP * SUB
            nhb = npar * GP * SUB
            nrow = pl.multiple_of(base_out + (g + 1) * GP * SUB, 8)

            @pl.when(g + 1 < ngroups)
            def _stage():
                pltpu.async_copy(osrc.at[pl.ds(nrow, GP * SUB)],
                                 sbv.at[pl.ds(nhb, GP * SUB)], esem)
                pltpu.async_copy(odst.at[pl.ds(nrow, GP * SUB)],
                                 dbv.at[pl.ds(nhb, GP * SUB)], esem)
                pltpu.async_copy(oval.at[pl.ds(nrow, GP * SUB)],
                                 vbv.at[pl.ds(nhb, GP * SUB)], esem)

            for p in range(GP):
                sl2 = (p + 2) % NSL
                if p == 0:
                    @pl.when(g >= 1)
                    def _drain0():
                        scatter_wait(sl2)
                    gather(sl2, hb + 2 * SUB)
                else:
                    @pl.when(g + 1 < ngroups)
                    def _pref():
                        scatter_wait(sl2)
                        if p == 1:
                            pltpu.make_async_copy(
                                osrc.at[pl.ds(nrow, GP * SUB)],
                                sbv.at[pl.ds(nhb, GP * SUB)], esem).wait()
                            pltpu.make_async_copy(
                                odst.at[pl.ds(nrow, GP * SUB)],
                                dbv.at[pl.ds(nhb, GP * SUB)], esem).wait()
                            pltpu.make_async_copy(
                                oval.at[pl.ds(nrow, GP * SUB)],
                                vbv.at[pl.ds(nhb, GP * SUB)], esem).wait()
                        gather(sl2, nhb + (p - 1) * SUB)

                gather_wait(p, hb + p * SUB)
                _scale_chunk(rows, vbv, hb + p * SUB, p)
                copy_dst(p, hb + p * SUB)
                scatter(p)
            return 0
        lax.fori_loop(0, ngroups, grp, 0)
        for sl in range(NSL):
            scatter_wait(sl)


def _writeback(acc, dst_tab, c, s):
    pltpu.sync_copy(
        acc.at[pl.ds(s * ROWS_PER_TILE, ROWS_PER_TILE)],
        dst_tab.at[pl.ds(c * HALF_PAD + s * ROWS_PER_TILE, ROWS_PER_TILE)])


def _body1(ego_tab, srcr, dstr, valr,
           t1, osrc, odst, oval, counts,
           acc, sb, db, vb, fs, fd, fv,
           sbv, dbv, vbv, dx0, dx1, dx2, rows, cntv,
           gs0, gs1, gs2, ss0, ss1, ss2, esem):
    c = lax.axis_index("c")
    s = lax.axis_index("s")
    tid = c * NS + s
    base_out = pl.multiple_of(tid * CAP, 8)
    dx = [dx0, dx1, dx2]
    gsem = [gs0, gs1, gs2]
    ssem = [ss0, ss1, ss2]
    dlo = c * HALF
    dhi = dlo + HALF

    def block(b, carry):
        pos, total = carry
        row0 = s * IN_CPT + b * 8
        pltpu.sync_copy(srcr.at[pl.ds(row0, 8)], sb)
        pltpu.sync_copy(dstr.at[pl.ds(row0, 8)], db)
        pltpu.sync_copy(valr.at[pl.ds(row0, 8)], vb)

        iota16 = lax.iota(jnp.int32, 16)

        def chunk(r, carry2):
            pos2, total2 = carry2

            @plsc.parallel_loop(0, 8, unroll=2, carry=pos2)
            def g16(u, pos3):
                sv = sb[r, pl.ds(u * 16, 16)]
                dv = db[r, pl.ds(u * 16, 16)]
                vv = vb[r, pl.ds(u * 16, 16)]
                m = (dv >= dlo) & (dv < dhi)
                mi = jnp.where(m, jnp.full((16,), 1, jnp.int32),
                               jnp.full((16,), 0, jnp.int32))
                incl = plsc.cumsum(mi)
                excl = incl - mi
                rsv = jnp.where(sv >= HALF, sv + (HALF_PAD - HALF), sv)
                ldv = dv - dlo
                idx = jnp.where(m, pos3 + excl, TRASH + iota16)
                plsc.store_scatter(fs, [idx], rsv)
                plsc.store_scatter(fd, [idx], ldv)
                plsc.store_scatter(fv, [idx], vv)
                return pos3 + incl[15]
            pos2 = g16

            do_flush = pos2 >= FLUSH

            @pl.when(do_flush)
            def _flush():
                pltpu.sync_copy(fs.at[pl.ds(0, FLUSH)],
                                osrc.at[pl.ds(pl.multiple_of(base_out + total2, 8), FLUSH)])
                pltpu.sync_copy(fd.at[pl.ds(0, FLUSH)],
                                odst.at[pl.ds(pl.multiple_of(base_out + total2, 8), FLUSH)])
                pltpu.sync_copy(fv.at[pl.ds(0, FLUSH)],
                                oval.at[pl.ds(pl.multiple_of(base_out + total2, 8), FLUSH)])
                ntail = (pos2 - FLUSH + 15) // 16

                def mv(t, _):
                    fs[pl.ds(t * 16, 16)] = fs[pl.ds(FLUSH + t * 16, 16)]
                    fd[pl.ds(t * 16, 16)] = fd[pl.ds(FLUSH + t * 16, 16)]
                    fv[pl.ds(t * 16, 16)] = fv[pl.ds(FLUSH + t * 16, 16)]
                    return 0
                lax.fori_loop(0, ntail, mv, 0)

            pos2 = jnp.where(do_flush, pos2 - FLUSH, pos2)
            total2 = jnp.where(do_flush, total2 + FLUSH, total2)
            return (pos2, total2)
        return lax.fori_loop(0, 8, chunk, (pos, total))

    pos, total = lax.fori_loop(0, IN_CPT // 8, block,
                               (jnp.int32(0), jnp.int32(0)))

    @pl.when(pos > 0)
    def _tail():
        zi = jnp.zeros((16,), jnp.int32)
        zf = jnp.zeros((16,), jnp.float32)
        fs[pl.ds(pos, 16)] = zi
        fd[pl.ds(pos, 16)] = zi
        fv[pl.ds(pos, 16)] = zf
        pos16 = (pos + 15) & ~jnp.int32(15)

        def padk(k, _):
            off = pos16 + k * 16
            fs[pl.ds(off, 16)] = zi
            fd[pl.ds(off, 16)] = zi
            fv[pl.ds(off, 16)] = zf
            return 0
        lax.fori_loop(0, (FLUSH - pos16) // 16, padk, 0)
        pltpu.sync_copy(fs.at[pl.ds(0, FLUSH)],
                        osrc.at[pl.ds(pl.multiple_of(base_out + total, 8), FLUSH)])
        pltpu.sync_copy(fd.at[pl.ds(0, FLUSH)],
                        odst.at[pl.ds(pl.multiple_of(base_out + total, 8), FLUSH)])
        pltpu.sync_copy(fv.at[pl.ds(0, FLUSH)],
                        oval.at[pl.ds(pl.multiple_of(base_out + total, 8), FLUSH)])

    total = jnp.where(pos > 0, total + FLUSH, total)
    cntv[pl.ds(0, 16)] = jnp.broadcast_to(total, (16,)).astype(jnp.int32)
    pltpu.sync_copy(cntv, counts.at[pl.ds(pl.multiple_of(tid * 16, 16), 16)])

    _zero_acc(acc, rows, s)
    plsc.subcore_barrier()
    _edges_phase(ego_tab, osrc, odst, oval, cntv, acc,
                 sbv, dbv, vbv, dx, rows, gsem, ssem, esem, base_out)
    plsc.subcore_barrier()
    _writeback(acc, t1, c, s)


def _body2(tin, osrc, odst, oval, counts,
           tout,
           acc, sbv, dbv, vbv, dx0, dx1, dx2, rows, cntv,
           gs0, gs1, gs2, ss0, ss1, ss2, esem):
    c = lax.axis_index("c")
    s = lax.axis_index("s")
    tid = c * NS + s
    dx = [dx0, dx1, dx2]
    gsem = [gs0, gs1, gs2]
    ssem = [ss0, ss1, ss2]
    pltpu.sync_copy(counts.at[pl.ds(pl.multiple_of(tid * 16, 16), 16)], cntv)
    _zero_acc(acc, rows, s)
    plsc.subcore_barrier()
    _edges_phase(tin, osrc, odst, oval, cntv, acc,
                 sbv, dbv, vbv, dx, rows, gsem, ssem, esem, tid * CAP)
    plsc.subcore_barrier()
    _writeback(acc, tout, c, s)


def _body3(tin, t1, osrc, odst, oval, counts,
           outf,
           acc, sbv, dbv, vbv, dx0, dx1, dx2, rows, cntv,
           gs0, gs1, gs2, ss0, ss1, ss2, esem):
    c = lax.axis_index("c")
    s = lax.axis_index("s")
    tid = c * NS + s
    dx = [dx0, dx1, dx2]
    gsem = [gs0, gs1, gs2]
    ssem = [ss0, ss1, ss2]
    pltpu.sync_copy(counts.at[pl.ds(pl.multiple_of(tid * 16, 16), 16)], cntv)
    _zero_acc(acc, rows, s)
    plsc.subcore_barrier()
    _edges_phase(tin, osrc, odst, oval, cntv, acc,
                 sbv, dbv, vbv, dx, rows, gsem, ssem, esem, tid * CAP)
    plsc.subcore_barrier()

    def fin(z, _):
        gbase = c * HALF_PAD + s * ROWS_PER_TILE + z * ZCH
        abase = s * ROWS_PER_TILE + z * ZCH
        pltpu.sync_copy(t1.at[pl.ds(gbase, ZCH)], rows.at[pl.ds(0, ZCH)])
        pltpu.sync_copy(tin.at[pl.ds(gbase, ZCH)], rows.at[pl.ds(ZCH, ZCH)])
        pltpu.sync_copy(acc.at[pl.ds(abase, ZCH)], rows.at[pl.ds(2 * ZCH, ZCH)])

        @plsc.parallel_loop(0, ZCH, unroll=2)
        def m(i):
            for o in (0, 16, 32, 48):
                x = (rows[i, pl.ds(o, 16)] + rows[ZCH + i, pl.ds(o, 16)]
                     + rows[2 * ZCH + i, pl.ds(o, 16)]) * jnp.float32(1.0 / 3.0)
                rows[i, pl.ds(o, 16)] = x
        pltpu.sync_copy(rows.at[pl.ds(0, ZCH)], outf.at[pl.ds(gbase, ZCH)])
        return 0
    lax.fori_loop(0, NZC, fin, 0)


_MESH = plsc.VectorSubcoreMesh(core_axis_name="c", subcore_axis_name="s")
_PARAMS = pltpu.CompilerParams(use_tc_tiling_on_sc=False,
                               needs_layout_passes=False)

_COMMON_SCRATCH = [
    pltpu.VMEM_SHARED((HALF_PAD, D), jnp.float32),
    pltpu.VMEM((2 * GP * SUB,), jnp.int32),
    pltpu.VMEM((2 * GP * SUB,), jnp.int32),
    pltpu.VMEM((2 * GP * SUB,), jnp.float32),
    pltpu.VMEM((SUB,), jnp.int32),
    pltpu.VMEM((SUB,), jnp.int32),
    pltpu.VMEM((SUB,), jnp.int32),
    pltpu.VMEM((NSL * SUB, D), jnp.float32),
    pltpu.VMEM((16,), jnp.int32),
] + [pltpu.SemaphoreType.DMA] * 7

_run1 = pl.kernel(
    _body1,
    out_type=(jax.ShapeDtypeStruct((NT, D), jnp.float32),
              jax.ShapeDtypeStruct((OSZ,), jnp.int32),
              jax.ShapeDtypeStruct((OSZ,), jnp.int32),
              jax.ShapeDtypeStruct((OSZ,), jnp.float32),
              jax.ShapeDtypeStruct((NC * NS * 16,), jnp.int32)),
    mesh=_MESH,
    compiler_params=_PARAMS,
    scratch_types=[
        pltpu.VMEM_SHARED((HALF_PAD, D), jnp.float32),
        pltpu.VMEM((8, 128), jnp.int32),
        pltpu.VMEM((8, 128), jnp.int32),
        pltpu.VMEM((8, 128), jnp.float32),
        pltpu.VMEM((FIFO,), jnp.int32),
        pltpu.VMEM((FIFO,), jnp.int32),
        pltpu.VMEM((FIFO,), jnp.float32),
        pltpu.VMEM((2 * GP * SUB,), jnp.int32),
        pltpu.VMEM((2 * GP * SUB,), jnp.int32),
        pltpu.VMEM((2 * GP * SUB,), jnp.float32),
        pltpu.VMEM((SUB,), jnp.int32),
        pltpu.VMEM((SUB,), jnp.int32),
        pltpu.VMEM((SUB,), jnp.int32),
        pltpu.VMEM((NSL * SUB, D), jnp.float32),
        pltpu.VMEM((16,), jnp.int32),
    ] + [pltpu.SemaphoreType.DMA] * 7,
)

_run2 = pl.kernel(
    _body2,
    out_type=jax.ShapeDtypeStruct((NT, D), jnp.float32),
    mesh=_MESH,
    compiler_params=_PARAMS,
    scratch_types=list(_COMMON_SCRATCH),
)

_run3 = pl.kernel(
    _body3,
    out_type=jax.ShapeDtypeStruct((NT, D), jnp.float32),
    mesh=_MESH,
    compiler_params=_PARAMS,
    scratch_types=list(_COMMON_SCRATCH),
)


def kernel(user_emb, item_emb, edge_val, edge_src, edge_dst):
    zpad = jnp.zeros((HALF_PAD - HALF, D), jnp.float32)
    ego_tab = jnp.concatenate([user_emb, zpad, item_emb, zpad], axis=0)
    e = edge_src.shape[0]
    pad = E_PAD - e
    srcr = jnp.pad(edge_src.astype(jnp.int32), (0, pad)).reshape(E_PAD // IN_W, IN_W)
    dstr = jnp.pad(edge_dst.astype(jnp.int32), (0, pad), constant_values=-1
                   ).reshape(E_PAD // IN_W, IN_W)
    valr = jnp.pad(edge_val, (0, pad)).reshape(E_PAD // IN_W, IN_W)
    t1, osrc, odst, oval, counts = _run1(ego_tab, srcr, dstr, valr)
    t2 = _run2(t1, osrc, odst, oval, counts)
    outf = _run3(t2, t1, osrc, odst, oval, counts)
    return outf[:N_USERS], outf[HALF_PAD:HALF_PAD + N_ITEMS]
